# baseline (speedup 1.0000x reference)
import functools

import jax
import jax.numpy as jnp
from jax import lax
from jax.experimental import pallas as pl
from jax.experimental.pallas import tpu as pltpu
from jax.experimental.pallas import tpu_sc as plsc

E = 8
T = 2048
D = 1024
FF = 1024
B = 256
G = (T * 2) // B + E
NSLOT = G * B
NW = 32
CHUNK = T // NW
SUB = 8
DH = D // 2
_HI = jax.lax.Precision.HIGHEST


def _routing_body(rl_ref, s1_ref, s2_ref, w1_ref, w2_ref, tent_ref):
    l = rl_ref[...]
    p = jnp.exp(l - jnp.max(l, axis=1, keepdims=True))
    pn = p / jnp.sum(p, axis=1, keepdims=True)
    iota = lax.broadcasted_iota(jnp.int32, (T, E), 1)
    m1 = jnp.max(pn, axis=1, keepdims=True)
    i1 = jnp.min(jnp.where(pn == m1, iota, E), axis=1, keepdims=True)
    pn2 = jnp.where(iota == i1, -1.0, pn)
    m2 = jnp.max(pn2, axis=1, keepdims=True)
    i2 = jnp.min(jnp.where(pn2 == m2, iota, E), axis=1, keepdims=True)
    sel1 = iota == i1
    sel2 = iota == i2
    ind = (sel1 | sel2).astype(jnp.float32)

    C = 256
    ls = (lax.broadcasted_iota(jnp.int32, (C, C), 1)
          < lax.broadcasted_iota(jnp.int32, (C, C), 0)).astype(jnp.float32)
    base = jnp.zeros((1, E), jnp.float32)
    chunks = []
    for c in range(T // C):
        indc = ind[c * C:(c + 1) * C, :]
        chunks.append(jnp.dot(ls, indc, precision=_HI) + base)
        base = base + jnp.sum(indc, axis=0, keepdims=True)
    pos = jnp.concatenate(chunks, axis=0)
    counts = base

    ceil_t = jnp.floor((counts + (B - 1)) / B)
    u8 = (lax.broadcasted_iota(jnp.int32, (E, E), 0)
          < lax.broadcasted_iota(jnp.int32, (E, E), 1)).astype(jnp.float32)
    tile_off = jnp.dot(ceil_t, u8, precision=_HI)
    slotmat = B * tile_off + pos

    s1 = jnp.sum(jnp.where(sel1, slotmat, 0.0), axis=1, keepdims=True)
    s2 = jnp.sum(jnp.where(sel2, slotmat, 0.0), axis=1, keepdims=True)
    s1_ref[...] = s1.astype(jnp.int32)
    s2_ref[...] = s2.astype(jnp.int32)
    denom = m1 + m2
    w1_ref[...] = m1 / denom
    w2_ref[...] = m2 / denom

    end = tile_off + ceil_t
    endmat = jnp.dot(jnp.ones((32, 1), jnp.float32), end, precision=_HI)
    gio = lax.broadcasted_iota(jnp.int32, (32, E), 0).astype(jnp.float32)
    te = jnp.sum(jnp.where(endmat <= gio, 1.0, 0.0), axis=1, keepdims=True)
    te = jnp.minimum(te, float(E - 1))
    ntmat = jnp.dot(jnp.ones((32, 1), jnp.float32),
                    jnp.sum(ceil_t, axis=1, keepdims=True), precision=_HI)
    rio = lax.broadcasted_iota(jnp.int32, (32, 1), 0)
    tent_ref[...] = jnp.where(rio == G, ntmat, te).astype(jnp.int32)


def _routing(router_logits):
    return pl.pallas_call(
        _routing_body,
        grid=(1,),
        in_specs=[pl.BlockSpec((T, E), lambda i: (0, 0))],
        out_specs=[
            pl.BlockSpec((T, 1), lambda i: (0, 0)),
            pl.BlockSpec((T, 1), lambda i: (0, 0)),
            pl.BlockSpec((T, 1), lambda i: (0, 0)),
            pl.BlockSpec((T, 1), lambda i: (0, 0)),
            pl.BlockSpec((32, 1), lambda i: (0, 0)),
        ],
        out_shape=[
            jax.ShapeDtypeStruct((T, 1), jnp.int32),
            jax.ShapeDtypeStruct((T, 1), jnp.int32),
            jax.ShapeDtypeStruct((T, 1), jnp.float32),
            jax.ShapeDtypeStruct((T, 1), jnp.float32),
            jax.ShapeDtypeStruct((32, 1), jnp.int32),
        ],
    )(router_logits)


def _dispatch_body(x_hbm, s1_hbm, s2_hbm, xs_hbm,
                   xrows_v, idx1_v, idx2_v, sem):
    wid = lax.axis_index("s") * 2 + lax.axis_index("c")
    rows = pl.ds(wid * CHUNK, CHUNK)
    cx = pltpu.async_copy(x_hbm.at[rows], xrows_v, sem)
    c1 = pltpu.async_copy(s1_hbm.at[rows], idx1_v, sem)
    c2 = pltpu.async_copy(s2_hbm.at[rows], idx2_v, sem)
    cx.wait(); c1.wait(); c2.wait()
    sa = pltpu.async_copy(xrows_v, xs_hbm.at[idx1_v], sem)
    sb = pltpu.async_copy(xrows_v, xs_hbm.at[idx2_v], sem)
    sa.wait(); sb.wait()


def _dispatch(x, s1, s2):
    mesh = plsc.VectorSubcoreMesh(core_axis_name="c", subcore_axis_name="s")
    f = pl.kernel(
        _dispatch_body,
        mesh=mesh,
        out_type=jax.ShapeDtypeStruct((NSLOT, DH), jnp.float32),
        scratch_types=[
            pltpu.VMEM((CHUNK, DH), jnp.float32),
            pltpu.VMEM((CHUNK,), jnp.int32),
            pltpu.VMEM((CHUNK,), jnp.int32),
            pltpu.SemaphoreType.DMA,
        ],
    )
    return f(x, s1, s2)


def _ffn_body(te_ref, nt_ref, xs_ref, w13_ref, w2_ref, ys_ref,
              w13b_ref, w2b_ref):
    g = pl.program_id(0)

    @pl.when(g < nt_ref[0])
    def _():
        changed = (g == 0) | (te_ref[g] != te_ref[jnp.maximum(g - 1, 0)])

        @pl.when(changed)
        def _():
            w13b_ref[...] = w13_ref[0].astype(jnp.bfloat16)
            w2b_ref[...] = w2_ref[0].astype(jnp.bfloat16)

        v = pltpu.bitcast(xs_ref[...], jnp.int32)
        xe = pltpu.bitcast(v << 16, jnp.float32).astype(jnp.bfloat16)
        xo = pltpu.bitcast(v & jnp.int32(-65536), jnp.float32).astype(jnp.bfloat16)
        w13b = w13b_ref[...]
        h = (lax.dot_general(xe, w13b[:, :DH], (((1,), (1,)), ((), ())),
                             preferred_element_type=jnp.float32)
             + lax.dot_general(xo, w13b[:, DH:], (((1,), (1,)), ((), ())),
                               preferred_element_type=jnp.float32))
        gate = h[:, :FF]
        up = h[:, FF:]
        act = gate * (1.0 / (1.0 + jnp.exp(-gate))) * up
        y = lax.dot_general(act.astype(jnp.bfloat16), w2b_ref[...],
                            (((1,), (1,)), ((), ())),
                            preferred_element_type=jnp.float32)
        ys_ref[...] = y


def _ffn(te, nt, xs, w13_weight, w2_weight):
    grid_spec = pltpu.PrefetchScalarGridSpec(
        num_scalar_prefetch=2,
        grid=(G,),
        in_specs=[
            pl.BlockSpec((B, DH), lambda g, te_r, nt_r: (g, 0)),
            pl.BlockSpec((1, 2 * FF, D), lambda g, te_r, nt_r: (te_r[g], 0, 0)),
            pl.BlockSpec((1, D, FF), lambda g, te_r, nt_r: (te_r[g], 0, 0)),
        ],
        out_specs=pl.BlockSpec((B, D), lambda g, te_r, nt_r: (g, 0)),
        scratch_shapes=[
            pltpu.VMEM((2 * FF, D), jnp.bfloat16),
            pltpu.VMEM((D, FF), jnp.bfloat16),
        ],
    )
    return pl.pallas_call(
        _ffn_body,
        grid_spec=grid_spec,
        out_shape=jax.ShapeDtypeStruct((NSLOT, D), jnp.float32),
        compiler_params=pltpu.CompilerParams(
            dimension_semantics=("arbitrary",),
        ),
    )(te, nt, xs, w13_weight, w2_weight)


def _combine_body(s1_hbm, s2_hbm, w1_hbm, w2_hbm, ys_hbm, out_hbm,
                  idx1_v, idx2_v,
                  r1a_v, r2a_v, w1a_v, w2a_v, sema,
                  r1b_v, r2b_v, w1b_v, w2b_v, semb):
    wid = lax.axis_index("s") * 2 + lax.axis_index("c")
    base = wid * CHUNK
    ci1 = pltpu.async_copy(s1_hbm.at[pl.ds(base, CHUNK)], idx1_v, sema)
    ci2 = pltpu.async_copy(s2_hbm.at[pl.ds(base, CHUNK)], idx2_v, sema)
    ci1.wait(); ci2.wait()

    def fire(s, r1_v, r2_v, w1_v, w2_v, sem):
        pltpu.async_copy(ys_hbm.at[idx1_v.at[pl.ds(s * SUB, SUB)]], r1_v, sem)
        pltpu.async_copy(ys_hbm.at[idx2_v.at[pl.ds(s * SUB, SUB)]], r2_v, sem)
        pltpu.async_copy(w1_hbm.at[pl.ds(base + s * SUB, SUB)], w1_v.at[pl.ds(0, SUB)], sem)
        pltpu.async_copy(w2_hbm.at[pl.ds(base + s * SUB, SUB)], w2_v.at[pl.ds(0, SUB)], sem)

    def drain(r1_v, r2_v, w1_v, w2_v, sem):
        pltpu.make_async_copy(ys_hbm.at[idx1_v.at[pl.ds(0, SUB)]], r1_v, sem).wait()
        pltpu.make_async_copy(ys_hbm.at[idx2_v.at[pl.ds(0, SUB)]], r2_v, sem).wait()
        pltpu.make_async_copy(w1_hbm.at[pl.ds(0, SUB)], w1_v.at[pl.ds(0, SUB)], sem).wait()
        pltpu.make_async_copy(w2_hbm.at[pl.ds(0, SUB)], w2_v.at[pl.ds(0, SUB)], sem).wait()

    def compute_store(s, r1_v, r2_v, w1_v, w2_v):
        wv1 = w1_v[...]
        wv2 = w2_v[...]
        for i in range(SUB):
            a = wv1[i]
            b = wv2[i]
            for k in range(D // 16):
                sl = pl.ds(k * 16, 16)
                r1_v[i, sl] = a * r1_v[i, sl] + b * r2_v[i, sl]
        pltpu.sync_copy(r1_v, out_hbm.at[pl.ds(base + s * SUB, SUB)])

    fire(0, r1a_v, r2a_v, w1a_v, w2a_v, sema)

    def body(j, carry):
        s0 = 2 * j
        fire(s0 + 1, r1b_v, r2b_v, w1b_v, w2b_v, semb)
        drain(r1a_v, r2a_v, w1a_v, w2a_v, sema)
        compute_store(s0, r1a_v, r2a_v, w1a_v, w2a_v)

        @pl.when(j < (CHUNK // SUB) // 2 - 1)
        def _():
            fire(s0 + 2, r1a_v, r2a_v, w1a_v, w2a_v, sema)

        drain(r1b_v, r2b_v, w1b_v, w2b_v, semb)
        compute_store(s0 + 1, r1b_v, r2b_v, w1b_v, w2b_v)
        return carry

    lax.fori_loop(0, (CHUNK // SUB) // 2, body, 0)


def _combine(s1, s2, w1, w2, ys):
    mesh = plsc.VectorSubcoreMesh(core_axis_name="c", subcore_axis_name="s")
    f = pl.kernel(
        _combine_body,
        mesh=mesh,
        out_type=jax.ShapeDtypeStruct((T, D), jnp.float32),
        scratch_types=[
            pltpu.VMEM((CHUNK,), jnp.int32),
            pltpu.VMEM((CHUNK,), jnp.int32),
            pltpu.VMEM((SUB, D), jnp.float32),
            pltpu.VMEM((SUB, D), jnp.float32),
            pltpu.VMEM((16,), jnp.float32),
            pltpu.VMEM((16,), jnp.float32),
            pltpu.SemaphoreType.DMA,
            pltpu.VMEM((SUB, D), jnp.float32),
            pltpu.VMEM((SUB, D), jnp.float32),
            pltpu.VMEM((16,), jnp.float32),
            pltpu.VMEM((16,), jnp.float32),
            pltpu.SemaphoreType.DMA,
        ],
    )
    return f(s1, s2, w1, w2, ys)


@jax.jit
def kernel(x, router_logits, w13_weight, w2_weight):
    s1c, s2c, w1c, w2c, tent = _routing(router_logits)
    s1 = s1c.reshape(T)
    s2 = s2c.reshape(T)
    w1 = w1c.reshape(T)
    w2 = w2c.reshape(T)
    te = tent[:G, 0]
    nt = tent[G:G + 1, 0]
    xp = jnp.swapaxes(x.reshape(T, 2, DH), 1, 2).astype(jnp.bfloat16)
    xb32 = lax.bitcast_convert_type(xp, jnp.float32)
    xs = _dispatch(xb32, s1, s2)
    ys = _ffn(te, nt, xs, w13_weight, w2_weight)
    return _combine(s1, s2, w1, w2, ys)


# speedup vs baseline: 1.1915x; 1.1915x over previous
import functools

import jax
import jax.numpy as jnp
from jax import lax
from jax.experimental import pallas as pl
from jax.experimental.pallas import tpu as pltpu
from jax.experimental.pallas import tpu_sc as plsc

E = 8
T = 2048
D = 1024
FF = 1024
B = 256
G = (T * 2) // B + E
NSLOT = G * B
NW = 32
CHUNK = T // NW
SUB = 8
DH = D // 2
_HI = jax.lax.Precision.HIGHEST


def _routing_body(rl_ref, s1_ref, s2_ref, w1_ref, w2_ref, tent_ref):
    l = rl_ref[...]
    p = jnp.exp(l - jnp.max(l, axis=1, keepdims=True))
    pn = p / jnp.sum(p, axis=1, keepdims=True)
    iota = lax.broadcasted_iota(jnp.int32, (T, E), 1)
    m1 = jnp.max(pn, axis=1, keepdims=True)
    i1 = jnp.min(jnp.where(pn == m1, iota, E), axis=1, keepdims=True)
    pn2 = jnp.where(iota == i1, -1.0, pn)
    m2 = jnp.max(pn2, axis=1, keepdims=True)
    i2 = jnp.min(jnp.where(pn2 == m2, iota, E), axis=1, keepdims=True)
    sel1 = iota == i1
    sel2 = iota == i2
    ind = (sel1 | sel2).astype(jnp.float32)

    C = 256
    ls = (lax.broadcasted_iota(jnp.int32, (C, C), 1)
          < lax.broadcasted_iota(jnp.int32, (C, C), 0)).astype(jnp.float32)
    base = jnp.zeros((1, E), jnp.float32)
    chunks = []
    for c in range(T // C):
        indc = ind[c * C:(c + 1) * C, :]
        chunks.append(jnp.dot(ls, indc, precision=_HI) + base)
        base = base + jnp.sum(indc, axis=0, keepdims=True)
    pos = jnp.concatenate(chunks, axis=0)
    counts = base

    ceil_t = jnp.floor((counts + (B - 1)) / B)
    u8 = (lax.broadcasted_iota(jnp.int32, (E, E), 0)
          < lax.broadcasted_iota(jnp.int32, (E, E), 1)).astype(jnp.float32)
    tile_off = jnp.dot(ceil_t, u8, precision=_HI)
    slotmat = B * tile_off + pos

    s1 = jnp.sum(jnp.where(sel1, slotmat, 0.0), axis=1, keepdims=True)
    s2 = jnp.sum(jnp.where(sel2, slotmat, 0.0), axis=1, keepdims=True)
    s1_ref[...] = s1.astype(jnp.int32)
    s2_ref[...] = s2.astype(jnp.int32)
    denom = m1 + m2
    w1_ref[...] = m1 / denom
    w2_ref[...] = m2 / denom

    end = tile_off + ceil_t
    endmat = jnp.dot(jnp.ones((32, 1), jnp.float32), end, precision=_HI)
    gio = lax.broadcasted_iota(jnp.int32, (32, E), 0).astype(jnp.float32)
    te = jnp.sum(jnp.where(endmat <= gio, 1.0, 0.0), axis=1, keepdims=True)
    te = jnp.minimum(te, float(E - 1))
    ntmat = jnp.dot(jnp.ones((32, 1), jnp.float32),
                    jnp.sum(ceil_t, axis=1, keepdims=True), precision=_HI)
    rio = lax.broadcasted_iota(jnp.int32, (32, 1), 0)
    tent_ref[...] = jnp.where(rio == G, ntmat, te).astype(jnp.int32)


def _routing(router_logits):
    return pl.pallas_call(
        _routing_body,
        grid=(1,),
        in_specs=[pl.BlockSpec((T, E), lambda i: (0, 0))],
        out_specs=[
            pl.BlockSpec((T, 1), lambda i: (0, 0)),
            pl.BlockSpec((T, 1), lambda i: (0, 0)),
            pl.BlockSpec((T, 1), lambda i: (0, 0)),
            pl.BlockSpec((T, 1), lambda i: (0, 0)),
            pl.BlockSpec((32, 1), lambda i: (0, 0)),
        ],
        out_shape=[
            jax.ShapeDtypeStruct((T, 1), jnp.int32),
            jax.ShapeDtypeStruct((T, 1), jnp.int32),
            jax.ShapeDtypeStruct((T, 1), jnp.float32),
            jax.ShapeDtypeStruct((T, 1), jnp.float32),
            jax.ShapeDtypeStruct((32, 1), jnp.int32),
        ],
    )(router_logits)


def _dispatch_body(x_hbm, s1_hbm, s2_hbm, xs_hbm,
                   xrows_v, idx1_v, idx2_v, sem):
    wid = lax.axis_index("s") * 2 + lax.axis_index("c")
    rows = pl.ds(wid * CHUNK, CHUNK)
    cx = pltpu.async_copy(x_hbm.at[rows], xrows_v, sem)
    c1 = pltpu.async_copy(s1_hbm.at[rows], idx1_v, sem)
    c2 = pltpu.async_copy(s2_hbm.at[rows], idx2_v, sem)
    cx.wait(); c1.wait(); c2.wait()
    sa = pltpu.async_copy(xrows_v, xs_hbm.at[idx1_v], sem)
    sb = pltpu.async_copy(xrows_v, xs_hbm.at[idx2_v], sem)
    sa.wait(); sb.wait()


def _dispatch(x, s1, s2):
    mesh = plsc.VectorSubcoreMesh(core_axis_name="c", subcore_axis_name="s")
    f = pl.kernel(
        _dispatch_body,
        mesh=mesh,
        out_type=jax.ShapeDtypeStruct((NSLOT, DH), jnp.float32),
        scratch_types=[
            pltpu.VMEM((CHUNK, DH), jnp.float32),
            pltpu.VMEM((CHUNK,), jnp.int32),
            pltpu.VMEM((CHUNK,), jnp.int32),
            pltpu.SemaphoreType.DMA,
        ],
    )
    return f(x, s1, s2)


def _ffn_body(te_ref, nt_ref, xs_ref, w13_ref, w2_ref, ys_ref,
              w13b_ref, w2b_ref):
    g = pl.program_id(0)

    @pl.when(g < nt_ref[0])
    def _():
        changed = (g == 0) | (te_ref[g] != te_ref[jnp.maximum(g - 1, 0)])

        @pl.when(changed)
        def _():
            w13b_ref[...] = w13_ref[0].astype(jnp.bfloat16)
            w2b_ref[...] = w2_ref[0].astype(jnp.bfloat16)

        v = pltpu.bitcast(xs_ref[...], jnp.int32)
        xe = pltpu.bitcast(v << 16, jnp.float32).astype(jnp.bfloat16)
        xo = pltpu.bitcast(v & jnp.int32(-65536), jnp.float32).astype(jnp.bfloat16)
        w13b = w13b_ref[...]
        h = (lax.dot_general(xe, w13b[:, :DH], (((1,), (1,)), ((), ())),
                             preferred_element_type=jnp.float32)
             + lax.dot_general(xo, w13b[:, DH:], (((1,), (1,)), ((), ())),
                               preferred_element_type=jnp.float32))
        gate = h[:, :FF]
        up = h[:, FF:]
        act = gate * (1.0 / (1.0 + jnp.exp(-gate))) * up
        y = lax.dot_general(act.astype(jnp.bfloat16), w2b_ref[...],
                            (((1,), (1,)), ((), ())),
                            preferred_element_type=jnp.float32)
        ys_ref[...] = y


def _ffn(te, nt, xs, w13_weight, w2_weight):
    grid_spec = pltpu.PrefetchScalarGridSpec(
        num_scalar_prefetch=2,
        grid=(G,),
        in_specs=[
            pl.BlockSpec((B, DH), lambda g, te_r, nt_r: (g, 0)),
            pl.BlockSpec((1, 2 * FF, D), lambda g, te_r, nt_r: (te_r[g], 0, 0)),
            pl.BlockSpec((1, D, FF), lambda g, te_r, nt_r: (te_r[g], 0, 0)),
        ],
        out_specs=pl.BlockSpec((B, D), lambda g, te_r, nt_r: (g, 0)),
        scratch_shapes=[
            pltpu.VMEM((2 * FF, D), jnp.bfloat16),
            pltpu.VMEM((D, FF), jnp.bfloat16),
        ],
    )
    return pl.pallas_call(
        _ffn_body,
        grid_spec=grid_spec,
        out_shape=jax.ShapeDtypeStruct((NSLOT, D), jnp.float32),
        compiler_params=pltpu.CompilerParams(
            dimension_semantics=("arbitrary",),
        ),
    )(te, nt, xs, w13_weight, w2_weight)


def _combine_body(s1_hbm, s2_hbm, w1_hbm, w2_hbm, ys_hbm, out_hbm,
                  idx1_v, idx2_v,
                  r1a_v, r2a_v, w1a_v, w2a_v, sema,
                  r1b_v, r2b_v, w1b_v, w2b_v, semb):
    wid = lax.axis_index("s") * 2 + lax.axis_index("c")
    base = wid * CHUNK
    ci1 = pltpu.async_copy(s1_hbm.at[pl.ds(base, CHUNK)], idx1_v, sema)
    ci2 = pltpu.async_copy(s2_hbm.at[pl.ds(base, CHUNK)], idx2_v, sema)
    ci1.wait(); ci2.wait()

    def fire(s, r1_v, r2_v, w1_v, w2_v, sem):
        pltpu.async_copy(ys_hbm.at[idx1_v.at[pl.ds(s * SUB, SUB)]], r1_v, sem)
        pltpu.async_copy(ys_hbm.at[idx2_v.at[pl.ds(s * SUB, SUB)]], r2_v, sem)
        pltpu.async_copy(w1_hbm.at[pl.ds(base + s * SUB, SUB)], w1_v.at[pl.ds(0, SUB)], sem)
        pltpu.async_copy(w2_hbm.at[pl.ds(base + s * SUB, SUB)], w2_v.at[pl.ds(0, SUB)], sem)

    def drain(r1_v, r2_v, w1_v, w2_v, sem):
        pltpu.make_async_copy(ys_hbm.at[idx1_v.at[pl.ds(0, SUB)]], r1_v, sem).wait()
        pltpu.make_async_copy(ys_hbm.at[idx2_v.at[pl.ds(0, SUB)]], r2_v, sem).wait()
        pltpu.make_async_copy(w1_hbm.at[pl.ds(0, SUB)], w1_v.at[pl.ds(0, SUB)], sem).wait()
        pltpu.make_async_copy(w2_hbm.at[pl.ds(0, SUB)], w2_v.at[pl.ds(0, SUB)], sem).wait()

    def compute_store(s, r1_v, r2_v, w1_v, w2_v):
        wv1 = w1_v[...]
        wv2 = w2_v[...]
        for i in range(SUB):
            a = wv1[i]
            b = wv2[i]
            for k in range(D // 16):
                sl = pl.ds(k * 16, 16)
                r1_v[i, sl] = a * r1_v[i, sl] + b * r2_v[i, sl]
        pltpu.sync_copy(r1_v, out_hbm.at[pl.ds(base + s * SUB, SUB)])

    fire(0, r1a_v, r2a_v, w1a_v, w2a_v, sema)

    def body(j, carry):
        s0 = 2 * j
        fire(s0 + 1, r1b_v, r2b_v, w1b_v, w2b_v, semb)
        drain(r1a_v, r2a_v, w1a_v, w2a_v, sema)
        compute_store(s0, r1a_v, r2a_v, w1a_v, w2a_v)

        @pl.when(j < (CHUNK // SUB) // 2 - 1)
        def _():
            fire(s0 + 2, r1a_v, r2a_v, w1a_v, w2a_v, sema)

        drain(r1b_v, r2b_v, w1b_v, w2b_v, semb)
        compute_store(s0 + 1, r1b_v, r2b_v, w1b_v, w2b_v)
        return carry

    lax.fori_loop(0, (CHUNK // SUB) // 2, body, 0)


def _combine(s1, s2, w1, w2, ys):
    mesh = plsc.VectorSubcoreMesh(core_axis_name="c", subcore_axis_name="s")
    f = pl.kernel(
        _combine_body,
        mesh=mesh,
        out_type=jax.ShapeDtypeStruct((T, D), jnp.float32),
        scratch_types=[
            pltpu.VMEM((CHUNK,), jnp.int32),
            pltpu.VMEM((CHUNK,), jnp.int32),
            pltpu.VMEM((SUB, D), jnp.float32),
            pltpu.VMEM((SUB, D), jnp.float32),
            pltpu.VMEM((16,), jnp.float32),
            pltpu.VMEM((16,), jnp.float32),
            pltpu.SemaphoreType.DMA,
            pltpu.VMEM((SUB, D), jnp.float32),
            pltpu.VMEM((SUB, D), jnp.float32),
            pltpu.VMEM((16,), jnp.float32),
            pltpu.VMEM((16,), jnp.float32),
            pltpu.SemaphoreType.DMA,
        ],
    )
    return f(s1, s2, w1, w2, ys)


@jax.jit
def kernel(x, router_logits, w13_weight, w2_weight):
    s1c, s2c, w1c, w2c, tent = _routing(router_logits)
    s1 = s1c.reshape(T)
    s2 = s2c.reshape(T)
    w1 = w1c.reshape(T)
    w2 = w2c.reshape(T)
    te = tent[:G, 0]
    nt = tent[G:G + 1, 0]
    lo = lax.bitcast_convert_type(x[:, :DH].astype(jnp.bfloat16),
                                  jnp.uint16).astype(jnp.uint32)
    hi = lax.bitcast_convert_type(x[:, DH:].astype(jnp.bfloat16),
                                  jnp.uint16).astype(jnp.uint32)
    xb32 = lax.bitcast_convert_type(lo | (hi << 16), jnp.float32)
    xs = _dispatch(xb32, s1, s2)
    ys = _ffn(te, nt, xs, w13_weight, w2_weight)
    return _combine(s1, s2, w1, w2, ys)


# speedup vs baseline: 1.2993x; 1.0905x over previous
import functools

import jax
import jax.numpy as jnp
from jax import lax
from jax.experimental import pallas as pl
from jax.experimental.pallas import tpu as pltpu
from jax.experimental.pallas import tpu_sc as plsc

E = 8
T = 2048
D = 1024
FF = 1024
B = 512
G = (T * 2) // B + E
NSLOT = G * B
NW = 32
CHUNK = T // NW
SUB = 8
_HI = jax.lax.Precision.HIGHEST


def _routing_body(rl_ref, s1_ref, s2_ref, w1_ref, w2_ref, tent_ref):
    l = rl_ref[...]
    p = jnp.exp(l - jnp.max(l, axis=1, keepdims=True))
    pn = p / jnp.sum(p, axis=1, keepdims=True)
    iota = lax.broadcasted_iota(jnp.int32, (T, E), 1)
    m1 = jnp.max(pn, axis=1, keepdims=True)
    i1 = jnp.min(jnp.where(pn == m1, iota, E), axis=1, keepdims=True)
    pn2 = jnp.where(iota == i1, -1.0, pn)
    m2 = jnp.max(pn2, axis=1, keepdims=True)
    i2 = jnp.min(jnp.where(pn2 == m2, iota, E), axis=1, keepdims=True)
    sel1 = iota == i1
    sel2 = iota == i2
    ind = (sel1 | sel2).astype(jnp.float32)

    C = 256
    ls = (lax.broadcasted_iota(jnp.int32, (C, C), 1)
          < lax.broadcasted_iota(jnp.int32, (C, C), 0)).astype(jnp.float32)
    base = jnp.zeros((1, E), jnp.float32)
    chunks = []
    for c in range(T // C):
        indc = ind[c * C:(c + 1) * C, :]
        chunks.append(jnp.dot(ls, indc, precision=_HI) + base)
        base = base + jnp.sum(indc, axis=0, keepdims=True)
    pos = jnp.concatenate(chunks, axis=0)
    counts = base

    ceil_t = jnp.floor((counts + (B - 1)) / B)
    u8 = (lax.broadcasted_iota(jnp.int32, (E, E), 0)
          < lax.broadcasted_iota(jnp.int32, (E, E), 1)).astype(jnp.float32)
    tile_off = jnp.dot(ceil_t, u8, precision=_HI)
    slotmat = B * tile_off + pos

    s1 = jnp.sum(jnp.where(sel1, slotmat, 0.0), axis=1, keepdims=True)
    s2 = jnp.sum(jnp.where(sel2, slotmat, 0.0), axis=1, keepdims=True)
    s1_ref[...] = s1.astype(jnp.int32)
    s2_ref[...] = s2.astype(jnp.int32)
    denom = m1 + m2
    w1_ref[...] = m1 / denom
    w2_ref[...] = m2 / denom

    end = tile_off + ceil_t
    endmat = jnp.dot(jnp.ones((32, 1), jnp.float32), end, precision=_HI)
    gio = lax.broadcasted_iota(jnp.int32, (32, E), 0).astype(jnp.float32)
    te = jnp.sum(jnp.where(endmat <= gio, 1.0, 0.0), axis=1, keepdims=True)
    te = jnp.minimum(te, float(E - 1))
    ntmat = jnp.dot(jnp.ones((32, 1), jnp.float32),
                    jnp.sum(ceil_t, axis=1, keepdims=True), precision=_HI)
    rio = lax.broadcasted_iota(jnp.int32, (32, 1), 0)
    tent_ref[...] = jnp.where(rio == G, ntmat, te).astype(jnp.int32)


def _routing(router_logits):
    return pl.pallas_call(
        _routing_body,
        grid=(1,),
        in_specs=[pl.BlockSpec((T, E), lambda i: (0, 0))],
        out_specs=[
            pl.BlockSpec((T, 1), lambda i: (0, 0)),
            pl.BlockSpec((T, 1), lambda i: (0, 0)),
            pl.BlockSpec((T, 1), lambda i: (0, 0)),
            pl.BlockSpec((T, 1), lambda i: (0, 0)),
            pl.BlockSpec((32, 1), lambda i: (0, 0)),
        ],
        out_shape=[
            jax.ShapeDtypeStruct((T, 1), jnp.int32),
            jax.ShapeDtypeStruct((T, 1), jnp.int32),
            jax.ShapeDtypeStruct((T, 1), jnp.float32),
            jax.ShapeDtypeStruct((T, 1), jnp.float32),
            jax.ShapeDtypeStruct((32, 1), jnp.int32),
        ],
    )(router_logits)


def _dispatch_body(x_hbm, s1_hbm, s2_hbm, xs_hbm,
                   xrows_v, idx1_v, idx2_v, sem):
    wid = lax.axis_index("s") * 2 + lax.axis_index("c")
    rows = pl.ds(wid * CHUNK, CHUNK)
    cx = pltpu.async_copy(x_hbm.at[rows], xrows_v, sem)
    c1 = pltpu.async_copy(s1_hbm.at[rows], idx1_v, sem)
    c2 = pltpu.async_copy(s2_hbm.at[rows], idx2_v, sem)
    cx.wait(); c1.wait(); c2.wait()
    sa = pltpu.async_copy(xrows_v, xs_hbm.at[idx1_v], sem)
    sb = pltpu.async_copy(xrows_v, xs_hbm.at[idx2_v], sem)
    sa.wait(); sb.wait()


def _dispatch(x, s1, s2):
    mesh = plsc.VectorSubcoreMesh(core_axis_name="c", subcore_axis_name="s")
    f = pl.kernel(
        _dispatch_body,
        mesh=mesh,
        out_type=jax.ShapeDtypeStruct((NSLOT, D), jnp.float32),
        scratch_types=[
            pltpu.VMEM((CHUNK, D), jnp.float32),
            pltpu.VMEM((CHUNK,), jnp.int32),
            pltpu.VMEM((CHUNK,), jnp.int32),
            pltpu.SemaphoreType.DMA,
        ],
    )
    return f(x, s1, s2)


def _ffn_body(te_ref, nt_ref, xs_ref, w13_ref, w2_ref, ys_ref):
    g = pl.program_id(0)

    @pl.when(g < nt_ref[0])
    def _():
        xs = xs_ref[...]
        h = lax.dot_general(xs, w13_ref[0], (((1,), (1,)), ((), ())),
                            preferred_element_type=jnp.float32)
        gate = h[:, :FF]
        up = h[:, FF:]
        act = gate * (1.0 / (1.0 + jnp.exp(-gate))) * up
        y = lax.dot_general(act, w2_ref[0], (((1,), (1,)), ((), ())),
                            preferred_element_type=jnp.float32)
        ys_ref[...] = y


def _ffn(te, nt, xs, w13_weight, w2_weight):
    grid_spec = pltpu.PrefetchScalarGridSpec(
        num_scalar_prefetch=2,
        grid=(G,),
        in_specs=[
            pl.BlockSpec((B, D), lambda g, te_r, nt_r: (g, 0)),
            pl.BlockSpec((1, 2 * FF, D), lambda g, te_r, nt_r: (te_r[g], 0, 0)),
            pl.BlockSpec((1, D, FF), lambda g, te_r, nt_r: (te_r[g], 0, 0)),
        ],
        out_specs=pl.BlockSpec((B, D), lambda g, te_r, nt_r: (g, 0)),
    )
    return pl.pallas_call(
        _ffn_body,
        grid_spec=grid_spec,
        out_shape=jax.ShapeDtypeStruct((NSLOT, D), jnp.float32),
        compiler_params=pltpu.CompilerParams(
            dimension_semantics=("arbitrary",),
        ),
    )(te, nt, xs, w13_weight, w2_weight)


def _combine_body(s1_hbm, s2_hbm, w1_hbm, w2_hbm, ys_hbm, out_hbm,
                  idx1_v, idx2_v,
                  r1a_v, r2a_v, w1a_v, w2a_v, sema,
                  r1b_v, r2b_v, w1b_v, w2b_v, semb):
    wid = lax.axis_index("s") * 2 + lax.axis_index("c")
    base = wid * CHUNK
    ci1 = pltpu.async_copy(s1_hbm.at[pl.ds(base, CHUNK)], idx1_v, sema)
    ci2 = pltpu.async_copy(s2_hbm.at[pl.ds(base, CHUNK)], idx2_v, sema)
    ci1.wait(); ci2.wait()

    def fire(s, r1_v, r2_v, w1_v, w2_v, sem):
        pltpu.async_copy(ys_hbm.at[idx1_v.at[pl.ds(s * SUB, SUB)]], r1_v, sem)
        pltpu.async_copy(ys_hbm.at[idx2_v.at[pl.ds(s * SUB, SUB)]], r2_v, sem)
        pltpu.async_copy(w1_hbm.at[pl.ds(base + s * SUB, SUB)], w1_v.at[pl.ds(0, SUB)], sem)
        pltpu.async_copy(w2_hbm.at[pl.ds(base + s * SUB, SUB)], w2_v.at[pl.ds(0, SUB)], sem)

    def drain(r1_v, r2_v, w1_v, w2_v, sem):
        pltpu.make_async_copy(ys_hbm.at[idx1_v.at[pl.ds(0, SUB)]], r1_v, sem).wait()
        pltpu.make_async_copy(ys_hbm.at[idx2_v.at[pl.ds(0, SUB)]], r2_v, sem).wait()
        pltpu.make_async_copy(w1_hbm.at[pl.ds(0, SUB)], w1_v.at[pl.ds(0, SUB)], sem).wait()
        pltpu.make_async_copy(w2_hbm.at[pl.ds(0, SUB)], w2_v.at[pl.ds(0, SUB)], sem).wait()

    def compute_store(s, r1_v, r2_v, w1_v, w2_v):
        wv1 = w1_v[...]
        wv2 = w2_v[...]
        for i in range(SUB):
            a = wv1[i]
            b = wv2[i]
            for k in range(D // 16):
                sl = pl.ds(k * 16, 16)
                r1_v[i, sl] = a * r1_v[i, sl] + b * r2_v[i, sl]
        pltpu.sync_copy(r1_v, out_hbm.at[pl.ds(base + s * SUB, SUB)])

    fire(0, r1a_v, r2a_v, w1a_v, w2a_v, sema)

    def body(j, carry):
        s0 = 2 * j
        fire(s0 + 1, r1b_v, r2b_v, w1b_v, w2b_v, semb)
        drain(r1a_v, r2a_v, w1a_v, w2a_v, sema)
        compute_store(s0, r1a_v, r2a_v, w1a_v, w2a_v)

        @pl.when(j < (CHUNK // SUB) // 2 - 1)
        def _():
            fire(s0 + 2, r1a_v, r2a_v, w1a_v, w2a_v, sema)

        drain(r1b_v, r2b_v, w1b_v, w2b_v, semb)
        compute_store(s0 + 1, r1b_v, r2b_v, w1b_v, w2b_v)
        return carry

    lax.fori_loop(0, (CHUNK // SUB) // 2, body, 0)


def _combine(s1, s2, w1, w2, ys):
    mesh = plsc.VectorSubcoreMesh(core_axis_name="c", subcore_axis_name="s")
    f = pl.kernel(
        _combine_body,
        mesh=mesh,
        out_type=jax.ShapeDtypeStruct((T, D), jnp.float32),
        scratch_types=[
            pltpu.VMEM((CHUNK,), jnp.int32),
            pltpu.VMEM((CHUNK,), jnp.int32),
            pltpu.VMEM((SUB, D), jnp.float32),
            pltpu.VMEM((SUB, D), jnp.float32),
            pltpu.VMEM((16,), jnp.float32),
            pltpu.VMEM((16,), jnp.float32),
            pltpu.SemaphoreType.DMA,
            pltpu.VMEM((SUB, D), jnp.float32),
            pltpu.VMEM((SUB, D), jnp.float32),
            pltpu.VMEM((16,), jnp.float32),
            pltpu.VMEM((16,), jnp.float32),
            pltpu.SemaphoreType.DMA,
        ],
    )
    return f(s1, s2, w1, w2, ys)


@jax.jit
def kernel(x, router_logits, w13_weight, w2_weight):
    s1c, s2c, w1c, w2c, tent = _routing(router_logits)
    s1 = s1c.reshape(T)
    s2 = s2c.reshape(T)
    w1 = w1c.reshape(T)
    w2 = w2c.reshape(T)
    te = tent[:G, 0]
    nt = tent[G:G + 1, 0]
    xs = _dispatch(x, s1, s2)
    ys = _ffn(te, nt, xs, w13_weight, w2_weight)
    return _combine(s1, s2, w1, w2, ys)
